# baseline (speedup 1.0000x reference)
import jax
import jax.numpy as jnp
from jax import lax
from jax.experimental import pallas as pl
from jax.experimental.pallas import tpu as pltpu

_B, _C, _D, _H, _W = 2, 768, 32, 32, 32
_SP = _D * _H * _W
_K = 192
_CC = 256


def _topk_body(s_ref, idx_ref):
    s = s_ref[...]
    rows = []
    for b in range(_B):
        sb = s[b]
        row = sb[None, :]
        chunks = []
        for ci in range(_C // _CC):
            scand = sb[ci * _CC:(ci + 1) * _CC][:, None]
            jj = lax.broadcasted_iota(jnp.int32, (_CC, _C), 1)
            cc = ci * _CC + lax.broadcasted_iota(jnp.int32, (_CC, _C), 0)
            beats = (row > scand) | ((row == scand) & (jj < cc))
            chunks.append(jnp.sum(beats.astype(jnp.int32), axis=1))
        rank_b = jnp.concatenate(chunks)

        kk = lax.broadcasted_iota(jnp.int32, (_K, _C), 0)
        c2 = lax.broadcasted_iota(jnp.int32, (_K, _C), 1)
        sel = (rank_b[None, :] == kk)
        rows.append(jnp.sum(jnp.where(sel, c2, 0), axis=1) + b * _C)
    idx_ref[...] = jnp.stack(rows)


_topk_call = pl.pallas_call(
    _topk_body,
    out_shape=jax.ShapeDtypeStruct((_B, _K), jnp.int32),
)


_WIN = 8


def _gather_body(idx_ref, x_ref, out_ref, sem):
    nb = _B * _K

    def issue(j):
        f = idx_ref[j]
        b = f // _C
        c = f - b * _C
        ob = j // _K
        ok = j - ob * _K
        pltpu.make_async_copy(x_ref.at[b, c], out_ref.at[ob, ok], sem).start()

    def drain(j):
        ob = j // _K
        ok = j - ob * _K
        pltpu.make_async_copy(x_ref.at[0, 0], out_ref.at[ob, ok], sem).wait()

    def body(j, carry):
        issue(j)

        @pl.when(j >= _WIN)
        def _():
            drain(j - _WIN)

        return carry

    lax.fori_loop(0, nb, body, 0)

    def tail(j, carry):
        drain(j)
        return carry

    lax.fori_loop(nb - _WIN, nb, tail, 0)


_gather_call = pl.pallas_call(
    _gather_body,
    grid_spec=pltpu.PrefetchScalarGridSpec(
        num_scalar_prefetch=1,
        grid=(1,),
        in_specs=[pl.BlockSpec(memory_space=pltpu.MemorySpace.HBM)],
        out_specs=pl.BlockSpec(memory_space=pltpu.MemorySpace.HBM),
        scratch_shapes=[pltpu.SemaphoreType.DMA],
    ),
    out_shape=jax.ShapeDtypeStruct((_B, _K, _D, _H, _W), jnp.float32),
)


def kernel(x, W1, b1, W2, b2):
    y = jnp.mean(x, axis=(2, 3, 4))
    h = y @ W1.T + b1
    h = jnp.where(h >= 0, h, 0.01 * h)
    h = h @ W2.T + b2
    s = jax.nn.sigmoid(h)
    idx = _topk_call(s)
    out = _gather_call(idx.reshape(-1), x)
    return out


# speedup vs baseline: 6.3161x; 6.3161x over previous
import jax
import jax.numpy as jnp
from jax import lax
from jax.experimental import pallas as pl
from jax.experimental.pallas import tpu as pltpu

_B, _C, _D, _H, _W = 2, 768, 32, 32, 32
_SP = _D * _H * _W
_K = 192
_CC = 256


def _topk_body(s_ref, idx_ref):
    s = s_ref[...]
    rows = []
    for b in range(_B):
        sb = s[b]
        row = sb[None, :]
        chunks = []
        for ci in range(_C // _CC):
            scand = sb[ci * _CC:(ci + 1) * _CC][:, None]
            jj = lax.broadcasted_iota(jnp.int32, (_CC, _C), 1)
            cc = ci * _CC + lax.broadcasted_iota(jnp.int32, (_CC, _C), 0)
            beats = (row > scand) | ((row == scand) & (jj < cc))
            chunks.append(jnp.sum(beats.astype(jnp.int32), axis=1))
        rank_b = jnp.concatenate(chunks)

        kk = lax.broadcasted_iota(jnp.int32, (_K, _C), 0)
        c2 = lax.broadcasted_iota(jnp.int32, (_K, _C), 1)
        sel = (rank_b[None, :] == kk)
        rows.append(jnp.sum(jnp.where(sel, c2, 0), axis=1) + b * _C)
    idx_ref[...] = jnp.stack(rows)


_topk_call = pl.pallas_call(
    _topk_body,
    out_shape=jax.ShapeDtypeStruct((_B, _K), jnp.int32),
)


def _gather_body(idx_ref, x_ref, out_ref):
    out_ref[...] = x_ref[...]


_gather_call = pl.pallas_call(
    _gather_body,
    grid_spec=pltpu.PrefetchScalarGridSpec(
        num_scalar_prefetch=1,
        grid=(_B * _K,),
        in_specs=[
            pl.BlockSpec(
                (1, 1, _D, _H, _W),
                lambda i, idx_ref: (idx_ref[i] // _C, idx_ref[i] % _C, 0, 0, 0),
            ),
        ],
        out_specs=pl.BlockSpec(
            (1, 1, _D, _H, _W),
            lambda i, idx_ref: (i // _K, i % _K, 0, 0, 0),
        ),
    ),
    out_shape=jax.ShapeDtypeStruct((_B, _K, _D, _H, _W), jnp.float32),
)


def kernel(x, W1, b1, W2, b2):
    y = jnp.mean(x, axis=(2, 3, 4))
    h = y @ W1.T + b1
    h = jnp.where(h >= 0, h, 0.01 * h)
    h = h @ W2.T + b2
    s = jax.nn.sigmoid(h)
    idx = _topk_call(s)
    out = _gather_call(idx.reshape(-1), x)
    return out


# speedup vs baseline: 13.4309x; 2.1265x over previous
import jax
import jax.numpy as jnp
from jax import lax
from jax.experimental import pallas as pl
from jax.experimental.pallas import tpu as pltpu

_B, _C, _D, _H, _W = 2, 768, 32, 32, 32
_SP = _D * _H * _W
_K = 192
_CC = 256


def _topk_body(s_ref, idx_ref):
    s = s_ref[...]
    rows = []
    for b in range(_B):
        sb = s[b]
        row = sb[None, :]
        chunks = []
        for ci in range(_C // _CC):
            scand = sb[ci * _CC:(ci + 1) * _CC][:, None]
            jj = lax.broadcasted_iota(jnp.int32, (_CC, _C), 1)
            cc = ci * _CC + lax.broadcasted_iota(jnp.int32, (_CC, _C), 0)
            beats = (row > scand) | ((row == scand) & (jj < cc))
            chunks.append(jnp.sum(beats.astype(jnp.int32), axis=1))
        rank_b = jnp.concatenate(chunks)

        kk = lax.broadcasted_iota(jnp.int32, (_K, _C), 0)
        c2 = lax.broadcasted_iota(jnp.int32, (_K, _C), 1)
        sel = (rank_b[None, :] == kk)
        rows.append(jnp.sum(jnp.where(sel, c2, 0), axis=1) + b * _C)
    idx_ref[...] = jnp.stack(rows)


_topk_call = pl.pallas_call(
    _topk_body,
    out_shape=jax.ShapeDtypeStruct((_B, _K), jnp.int32),
)


def _gather_body(idx_ref, x_ref, out_ref):
    out_ref[...] = x_ref[...]


_gather_call = pl.pallas_call(
    _gather_body,
    grid_spec=pltpu.PrefetchScalarGridSpec(
        num_scalar_prefetch=1,
        grid=(_B * _K,),
        in_specs=[
            pl.BlockSpec(
                (1, 1, _SP // 128, 128),
                lambda i, idx_ref: (idx_ref[i] // _C, idx_ref[i] % _C, 0, 0),
            ),
        ],
        out_specs=pl.BlockSpec(
            (1, 1, _SP // 128, 128),
            lambda i, idx_ref: (i // _K, i % _K, 0, 0),
        ),
    ),
    out_shape=jax.ShapeDtypeStruct((_B, _K, _SP // 128, 128), jnp.float32),
)


def kernel(x, W1, b1, W2, b2):
    y = jnp.mean(x, axis=(2, 3, 4))
    h = y @ W1.T + b1
    h = jnp.where(h >= 0, h, 0.01 * h)
    h = h @ W2.T + b2
    s = jax.nn.sigmoid(h)
    idx = _topk_call(s)
    out = _gather_call(idx.reshape(-1), x.reshape(_B, _C, _SP // 128, 128))
    return out.reshape(_B, _K, _D, _H, _W)
